# Initial kernel scaffold; baseline (speedup 1.0000x reference)
import jax
import jax.numpy as jnp
from jax.experimental import pallas as pl


def kernel(batch_points, W1, b1, W2, b2, W3, b3, Wm1, bm1, Wm2, bm2):
    raise NotImplementedError("write your pallas kernel here")


# speedup vs baseline: 67.7766x; 67.7766x over previous
import jax
import jax.numpy as jnp
from jax.experimental import pallas as pl
from jax.experimental.pallas import tpu as pltpu

_B, _N, _K = 4, 512, 8
_HID = 64


def _t(x):
    n = x.shape[1]
    eye = (jax.lax.broadcasted_iota(jnp.int32, (n, n), 0)
           == jax.lax.broadcasted_iota(jnp.int32, (n, n), 1)).astype(jnp.float32)
    return jax.lax.dot_general(eye, x, (((1,), (1,)), ((), ())),
                               preferred_element_type=jnp.float32)


def _body(pts_ref, ptst_ref, w1_ref, b1_ref, w2_ref, b2_ref, w3_ref, b3_ref,
          wm1a_ref, wm1b_ref, bm1_ref, wm2_ref, bm2_ref, out_ref):
    f32 = jnp.float32
    px_c = pts_ref[0, :, 0:1]
    py_c = pts_ref[0, :, 1:2]
    px_r = ptst_ref[0, 0:1, :]
    py_r = ptst_ref[0, 1:2, :]

    dx = px_c - px_r
    dy = py_c - py_r
    d2 = dx * dx + dy * dy

    col = jax.lax.broadcasted_iota(jnp.int32, (_N, _N), 1)
    big_idx = jnp.int32(_N)
    inf = f32(jnp.inf)

    S = jnp.zeros((_N, _N), f32)
    D = d2
    for t in range(_K + 1):
        rmin = jnp.min(D, axis=1, keepdims=True)
        cand = jnp.where(D == rmin, col, big_idx)
        first = jnp.min(cand, axis=1, keepdims=True)
        onehot = col == first
        if t > 0:
            S = S + onehot.astype(f32)
        D = jnp.where(onehot, inf, D)

    St = _t(S)
    indeg = jnp.sum(St, axis=1, keepdims=True)
    dis = jax.lax.rsqrt(indeg + f32(_K + 1))
    M = S + St

    def dot(a, b):
        return jax.lax.dot_general(a, b, (((1,), (0,)), ((), ())),
                                   preferred_element_type=f32)

    y = px_c * w1_ref[0:1, :] + py_c * w1_ref[1:2, :]
    z = dis * y
    x = jnp.maximum(dis * dot(M, z) + dis * dis * y + b1_ref[...], 0.0)
    for w_ref, b_ref in ((w2_ref, b2_ref), (w3_ref, b3_ref)):
        y = dot(x, w_ref[...])
        z = dis * y
        x = jnp.maximum(dis * dot(M, z) + dis * dis * y + b_ref[...], 0.0)

    A = dot(x, wm1a_ref[...]) + bm1_ref[...]
    At = _t(A)
    Ct = _t(dot(x, wm1b_ref[...]))
    wm2 = wm2_ref[...]

    acc = jnp.zeros((_N, _N), f32)
    CH = 8
    for c in range(_HID // CH):
        a3 = At[c * CH:(c + 1) * CH, :].reshape(CH, _N, 1)
        c3 = Ct[c * CH:(c + 1) * CH, :].reshape(CH, 1, _N)
        w3 = wm2[c * CH:(c + 1) * CH, :].reshape(CH, 1, 1)
        acc = acc + jnp.sum(jnp.maximum(a3 + c3, 0.0) * w3, axis=0)

    logits = acc + bm2_ref[...]
    prob = 1.0 / (1.0 + jnp.exp(-logits))
    row = jax.lax.broadcasted_iota(jnp.int32, (_N, _N), 0)
    upper = jnp.where(col > row, prob, 0.0)
    out_ref[0] = upper + _t(upper)


def kernel(batch_points, W1, b1, W2, b2, W3, b3, Wm1, bm1, Wm2, bm2):
    pts = batch_points.astype(jnp.float32)
    ptst = jnp.transpose(pts, (0, 2, 1))
    full = lambda shape: pl.BlockSpec(shape, lambda b: (0,) * len(shape))
    grid_spec = pl.GridSpec(
        grid=(_B,),
        in_specs=[
            pl.BlockSpec((1, _N, 2), lambda b: (b, 0, 0)),
            pl.BlockSpec((1, 2, _N), lambda b: (b, 0, 0)),
            full((2, _HID)), full((1, _HID)),
            full((_HID, _HID)), full((1, _HID)),
            full((_HID, _HID)), full((1, _HID)),
            full((_HID, _HID)), full((_HID, _HID)), full((1, _HID)),
            full((_HID, 1)), full((1, 1)),
        ],
        out_specs=pl.BlockSpec((1, _N, _N), lambda b: (b, 0, 0)),
    )
    return pl.pallas_call(
        _body,
        grid_spec=grid_spec,
        out_shape=jax.ShapeDtypeStruct((_B, _N, _N), jnp.float32),
    )(pts, ptst, W1, b1.reshape(1, _HID), W2, b2.reshape(1, _HID),
      W3, b3.reshape(1, _HID), Wm1[:_HID], Wm1[_HID:], bm1.reshape(1, _HID),
      Wm2, bm2.reshape(1, 1))
